# Initial kernel scaffold; baseline (speedup 1.0000x reference)
import jax
import jax.numpy as jnp
from jax.experimental import pallas as pl


def kernel(x):
    raise NotImplementedError("write your pallas kernel here")


# speedup vs baseline: 83.8875x; 83.8875x over previous
import functools

import jax
import jax.numpy as jnp
from jax.experimental import pallas as pl

_DOWNSAMPLE = 4


def _segmax_body(x_ref, sidc_ref, sidr_ref, endr_ref, out_ref, *, T, P):
    xv = x_ref[0]
    sid = sidc_ref[0]
    m = xv
    d = 1
    while d < T:
        sid_sh = jnp.concatenate(
            [jnp.full((d, 1), -1, jnp.int32), sid[:-d]], axis=0)
        m_sh = jnp.concatenate([m[:d], m[:-d]], axis=0)
        same = sid_sh == sid
        m = jnp.where(same, jnp.maximum(m, m_sh), m)
        d *= 2
    sid_row = sidr_ref[0, 0:1, :]
    end_row = endr_ref[0, 0:1, :]
    pids = jax.lax.broadcasted_iota(jnp.int32, (P, T), 0)
    onehot = ((pids == sid_row) & (end_row == 1)).astype(jnp.float32)
    out_ref[0] = jax.lax.dot(
        onehot, m,
        precision=jax.lax.Precision.HIGHEST,
        preferred_element_type=jnp.float32)


def _segment_max(x, sid, is_end, *, interpret=False):
    B, T, F = x.shape
    P = T // _DOWNSAMPLE
    sid_col = sid[:, :, None]
    sid_row = jnp.broadcast_to(sid[:, None, :], (B, 8, T))
    end_row = jnp.broadcast_to(is_end[:, None, :], (B, 8, T))
    return pl.pallas_call(
        functools.partial(_segmax_body, T=T, P=P),
        grid=(B,),
        in_specs=[
            pl.BlockSpec((1, T, F), lambda b: (b, 0, 0)),
            pl.BlockSpec((1, T, 1), lambda b: (b, 0, 0)),
            pl.BlockSpec((1, 8, T), lambda b: (b, 0, 0)),
            pl.BlockSpec((1, 8, T), lambda b: (b, 0, 0)),
        ],
        out_specs=pl.BlockSpec((1, P, F), lambda b: (b, 0, 0)),
        out_shape=jax.ShapeDtypeStruct((B, P, F), jnp.float32),
        interpret=interpret,
    )(x, sid_col, sid_row, end_row)


def kernel(x):
    B, T, F = x.shape
    npoints = T // _DOWNSAMPLE
    aux1 = x[:, : T - 1, :]
    aux2 = x[:, 1:, :]
    aux1E = jnp.sum(aux1 * aux1, axis=2)
    aux2E = jnp.sum(aux2 * aux2, axis=2)
    dif = aux2E - aux1E
    dif_conc = jnp.concatenate([jnp.zeros((B, 1), jnp.float32), dif], axis=1)
    LT = jnp.cumsum(jnp.abs(dif_conc), axis=1)
    LT_norm = LT / LT[:, -1:]
    LT_dif = LT_norm[:, 1:] - LT_norm[:, :-1]
    _vals, indices = jax.lax.top_k(LT_dif, npoints - 1)
    rows = jnp.arange(B, dtype=jnp.int32)[:, None]
    whichs = (
        jnp.zeros((B, T - 1), jnp.float32).at[rows, indices].set(1.0))
    index_points = jnp.cumsum(
        jnp.concatenate([jnp.zeros((B, 1), jnp.float32), whichs], axis=1),
        axis=1).astype(jnp.int32)
    sid = index_points
    is_end = jnp.concatenate(
        [(sid[:, 1:] > sid[:, :-1]).astype(jnp.int32),
         jnp.ones((B, 1), jnp.int32)], axis=1)
    return _segment_max(x, sid, is_end)
